# Initial kernel scaffold; baseline (speedup 1.0000x reference)
import jax
import jax.numpy as jnp
from jax.experimental import pallas as pl


def kernel(x, Wexp, bexp, gn_w, gn_b, rW1, rb1, rW2, rb2, merge_w, merge_b):
    raise NotImplementedError("write your pallas kernel here")


# speedup vs baseline: 1.3538x; 1.3538x over previous
import functools
import numpy as np
import jax
import jax.numpy as jnp
from jax.experimental import pallas as pl
from jax.experimental.pallas import tpu as pltpu

E = 8
C = 96
HID = 256
B = 8
H = 56
W = 56
N = H * W
EC = E * C
K9 = 9 * C
G = 8
CG = C // G
MT = 392
NSUB = N // MT
EPS = 1e-5


def _pce_body(xcol_ref, wcol_ref, xb_ref, brow_ref, gnw_ref, gnb_ref,
              rw1_ref, rb1_ref, rw2_ref, rb2_ref, mw_ref, mb_ref,
              m768_ref, m64e_ref, m96_ref, m8e_ref,
              out_ref, y_scr, acc_scr):
    wcol = wcol_ref[...]

    s = jnp.zeros((1, EC), jnp.float32)
    q = jnp.zeros((1, EC), jnp.float32)
    for i in range(NSUB):
        xt = xcol_ref[0, i * MT:(i + 1) * MT, :]
        yt = jnp.dot(xt, wcol, preferred_element_type=jnp.float32)
        y_scr[i * MT:(i + 1) * MT, :] = yt
        s = s + jnp.sum(yt, axis=0, keepdims=True)
        q = q + jnp.sum(yt * yt, axis=0, keepdims=True)

    brow = brow_ref[...]
    s2 = s + N * brow
    q2 = q + 2.0 * brow * s + N * brow * brow
    gs = jnp.dot(s2, m768_ref[...])
    gq = jnp.dot(q2, m768_ref[...])
    cnt = float(N * CG)
    mu = gs / cnt
    var = gq / cnt - mu * mu
    inv = jax.lax.rsqrt(var + EPS)
    mu_c = jnp.dot(mu, m64e_ref[...])
    inv_c = jnp.dot(inv, m64e_ref[...])
    gnw = gnw_ref[...]
    A = inv_c * gnw
    Bc = (brow - mu_c) * inv_c * gnw + gnb_ref[...]

    g = jnp.zeros((1, C), jnp.float32)
    for i in range(NSUB):
        g = g + jnp.sum(xb_ref[0, i * MT:(i + 1) * MT, :], axis=0,
                        keepdims=True)
    g = g / float(N)
    h1 = jnp.maximum(jnp.dot(g, rw1_ref[...]) + rb1_ref[...], 0.0)
    lg = jnp.dot(h1, rw2_ref[...]) + rb2_ref[...]
    lg = lg - jnp.max(lg, axis=-1, keepdims=True)
    ew = jnp.exp(lg)
    wts = ew / jnp.sum(ew, axis=-1, keepdims=True)
    sw = jnp.sum(wts, axis=-1, keepdims=True)

    ms = jnp.zeros((1, C), jnp.float32)
    mq = jnp.zeros((1, C), jnp.float32)
    for i in range(NSUB):
        acc = xb_ref[0, i * MT:(i + 1) * MT, :] * sw
        for e in range(E):
            a = y_scr[i * MT:(i + 1) * MT, e * C:(e + 1) * C]
            a = a * A[:, e * C:(e + 1) * C] + Bc[:, e * C:(e + 1) * C]
            a = jnp.maximum(a, 0.0)
            acc = acc + a * wts[:, e:e + 1]
        acc_scr[i * MT:(i + 1) * MT, :] = acc
        ms = ms + jnp.sum(acc, axis=0, keepdims=True)
        mq = mq + jnp.sum(acc * acc, axis=0, keepdims=True)

    gs2 = jnp.dot(ms, m96_ref[...])
    gq2 = jnp.dot(mq, m96_ref[...])
    mu2 = gs2 / cnt
    var2 = gq2 / cnt - mu2 * mu2
    inv2 = jax.lax.rsqrt(var2 + EPS)
    mu2_c = jnp.dot(mu2, m8e_ref[...])
    inv2_c = jnp.dot(inv2, m8e_ref[...])
    A2 = inv2_c * mw_ref[...]
    B2 = mb_ref[...] - mu2_c * A2
    for i in range(NSUB):
        out_ref[0, i * MT:(i + 1) * MT, :] = (
            acc_scr[i * MT:(i + 1) * MT, :] * A2 + B2)


@functools.partial(jax.jit, static_argnums=())
def kernel(x, Wexp, bexp, gn_w, gn_b, rW1, rb1, rW2, rb2, merge_w, merge_b):
    xt = jnp.transpose(x, (0, 2, 3, 1))
    xp = jnp.pad(xt, ((0, 0), (1, 1), (1, 1), (0, 0)))
    cols = [xp[:, ky:ky + H, kx:kx + W, :]
            for ky in range(3) for kx in range(3)]
    xcol = jnp.concatenate(cols, axis=-1).reshape(B, N, K9)
    xcol = xcol.astype(jnp.bfloat16)
    xb = xt.reshape(B, N, C)
    wcol = jnp.transpose(Wexp, (3, 4, 2, 0, 1)).reshape(K9, EC)
    wcol = wcol.astype(jnp.bfloat16)

    brow = bexp.reshape(1, EC)
    gnw_row = gn_w.reshape(1, EC)
    gnb_row = gn_b.reshape(1, EC)
    rb1_row = rb1.reshape(1, HID)
    rb2_row = rb2.reshape(1, E)
    mw_row = merge_w.reshape(1, C)
    mb_row = merge_b.reshape(1, C)

    cidx = np.arange(EC)
    gidx = (cidx // C) * G + (cidx % C) // CG
    m768 = (gidx[:, None] == np.arange(E * G)[None, :]).astype(np.float32)
    m64e = m768.T.copy()
    c96 = np.arange(C)
    m96 = ((c96 // CG)[:, None] == np.arange(G)[None, :]).astype(np.float32)
    m8e = m96.T.copy()

    const = lambda b: (0, 0)
    out = pl.pallas_call(
        _pce_body,
        grid=(B,),
        in_specs=[
            pl.BlockSpec((1, N, K9), lambda b: (b, 0, 0)),
            pl.BlockSpec((K9, EC), const),
            pl.BlockSpec((1, N, C), lambda b: (b, 0, 0)),
            pl.BlockSpec((1, EC), const),
            pl.BlockSpec((1, EC), const),
            pl.BlockSpec((1, EC), const),
            pl.BlockSpec((C, HID), const),
            pl.BlockSpec((1, HID), const),
            pl.BlockSpec((HID, E), const),
            pl.BlockSpec((1, E), const),
            pl.BlockSpec((1, C), const),
            pl.BlockSpec((1, C), const),
            pl.BlockSpec((EC, E * G), const),
            pl.BlockSpec((E * G, EC), const),
            pl.BlockSpec((C, G), const),
            pl.BlockSpec((G, C), const),
        ],
        out_specs=pl.BlockSpec((1, N, C), lambda b: (b, 0, 0)),
        out_shape=jax.ShapeDtypeStruct((B, N, C), jnp.float32),
        scratch_shapes=[
            pltpu.VMEM((N, EC), jnp.float32),
            pltpu.VMEM((N, C), jnp.float32),
        ],
    )(xcol, wcol, xb, brow, gnw_row, gnb_row, rW1, rb1_row, rW2, rb2_row,
      mw_row, mb_row, jnp.asarray(m768), jnp.asarray(m64e),
      jnp.asarray(m96), jnp.asarray(m8e))

    return jnp.transpose(out.reshape(B, H, W, C), (0, 3, 1, 2))


# speedup vs baseline: 1.8317x; 1.3530x over previous
import numpy as np
import jax
import jax.numpy as jnp
from jax.experimental import pallas as pl
from jax.experimental.pallas import tpu as pltpu

E = 8
C = 96
HID = 256
B = 8
H = 56
W = 56
N = H * W
NP = 58 * 56
EC = E * C
K3 = 3 * C
G = 8
CG = C // G
MT = 392
NSUB = N // MT
EPS = 1e-5


def kernel(x, Wexp, bexp, gn_w, gn_b, rW1, rb1, rW2, rb2, merge_w, merge_b):
    xt = jnp.transpose(x, (0, 2, 3, 1))
    xp = jnp.pad(xt, ((0, 0), (1, 1), (1, 1), (0, 0)))
    f3 = jnp.concatenate([xp[:, :, k:k + W, :] for k in range(3)],
                         axis=-1)
    f3 = f3.reshape(B, NP, K3).astype(jnp.bfloat16)
    xb = xt.reshape(B, N, C)
    wcol = jnp.transpose(Wexp, (3, 4, 2, 0, 1)).reshape(9 * C, EC)
    wcol = wcol.astype(jnp.bfloat16)

    brow = bexp.reshape(1, EC)
    gnw_row = gn_w.reshape(1, EC)
    gnb_row = gn_b.reshape(1, EC)
    rb1_row = rb1.reshape(1, HID)
    rb2_row = rb2.reshape(1, E)
    mw_row = merge_w.reshape(1, C)
    mb_row = merge_b.reshape(1, C)

    cidx = np.arange(EC)
    gidx = (cidx // C) * G + (cidx % C) // CG
    m768 = (gidx[:, None] == np.arange(E * G)[None, :]).astype(np.float32)
    m64e = m768.T.copy()
    c96 = np.arange(C)
    m96 = ((c96 // CG)[:, None] == np.arange(G)[None, :]).astype(np.float32)
    m8e = m96.T.copy()
    msel = ((cidx % C)[:, None] == c96[None, :]).astype(np.float32)
    mexp = ((cidx // C)[:, None] == np.arange(E)[None, :]).astype(np.float32)

    const = lambda b: (0, 0)
    out = pl.pallas_call(
        _pce_body,
        grid=(B,),
        in_specs=[
            pl.BlockSpec((1, NP, K3), lambda b: (b, 0, 0)),
            pl.BlockSpec((9 * C, EC), const),
            pl.BlockSpec((1, N, C), lambda b: (b, 0, 0)),
            pl.BlockSpec((1, EC), const),
            pl.BlockSpec((1, EC), const),
            pl.BlockSpec((1, EC), const),
            pl.BlockSpec((C, HID), const),
            pl.BlockSpec((1, HID), const),
            pl.BlockSpec((HID, E), const),
            pl.BlockSpec((1, E), const),
            pl.BlockSpec((1, C), const),
            pl.BlockSpec((1, C), const),
            pl.BlockSpec((EC, E * G), const),
            pl.BlockSpec((E * G, EC), const),
            pl.BlockSpec((C, G), const),
            pl.BlockSpec((G, C), const),
            pl.BlockSpec((EC, C), const),
            pl.BlockSpec((EC, E), const),
        ],
        out_specs=pl.BlockSpec((1, N, C), lambda b: (b, 0, 0)),
        out_shape=jax.ShapeDtypeStruct((B, N, C), jnp.float32),
        scratch_shapes=[
            pltpu.VMEM((N, EC), jnp.bfloat16),
            pltpu.VMEM((N, C), jnp.float32),
        ],
    )(f3, wcol, xb, brow, gnw_row, gnb_row, rW1, rb1_row, rW2, rb2_row,
      mw_row, mb_row, jnp.asarray(m768), jnp.asarray(m64e),
      jnp.asarray(m96), jnp.asarray(m8e), jnp.asarray(msel),
      jnp.asarray(mexp))

    return jnp.transpose(out.reshape(B, H, W, C), (0, 3, 1, 2))


def _pce_body(f3_ref, wcol_ref, xb_ref, brow_ref, gnw_ref, gnb_ref,
               rw1_ref, rb1_ref, rw2_ref, rb2_ref, mw_ref, mb_ref,
               m768_ref, m64e_ref, m96_ref, m8e_ref, msel_ref, mexp_ref,
               out_ref, y_scr, acc_scr):
    wcol = wcol_ref[...]
    w3 = [wcol[ky * K3:(ky + 1) * K3, :] for ky in range(3)]
    ones_mt = jnp.ones((1, MT), jnp.float32)

    s = jnp.zeros((1, EC), jnp.float32)
    q = jnp.zeros((1, EC), jnp.float32)
    for i in range(NSUB):
        yt = jnp.dot(f3_ref[0, i * MT:i * MT + MT, :], w3[0],
                     preferred_element_type=jnp.float32)
        yt = yt + jnp.dot(f3_ref[0, 56 + i * MT:56 + i * MT + MT, :], w3[1],
                          preferred_element_type=jnp.float32)
        yt = yt + jnp.dot(f3_ref[0, 112 + i * MT:112 + i * MT + MT, :], w3[2],
                          preferred_element_type=jnp.float32)
        y_scr[i * MT:(i + 1) * MT, :] = yt.astype(jnp.bfloat16)
        s = s + jnp.dot(ones_mt, yt, preferred_element_type=jnp.float32)
        q = q + jnp.dot(ones_mt, yt * yt,
                        preferred_element_type=jnp.float32)

    brow = brow_ref[...]
    s2 = s + N * brow
    q2 = q + 2.0 * brow * s + N * brow * brow
    gs = jnp.dot(s2, m768_ref[...])
    gq = jnp.dot(q2, m768_ref[...])
    cnt = float(N * CG)
    mu = gs / cnt
    var = gq / cnt - mu * mu
    inv = jax.lax.rsqrt(var + EPS)
    mu_c = jnp.dot(mu, m64e_ref[...])
    inv_c = jnp.dot(inv, m64e_ref[...])
    gnw = gnw_ref[...]
    A = inv_c * gnw
    Bc = (brow - mu_c) * inv_c * gnw + gnb_ref[...]

    g = jnp.zeros((1, C), jnp.float32)
    for i in range(NSUB):
        g = g + jnp.dot(ones_mt, xb_ref[0, i * MT:(i + 1) * MT, :],
                        preferred_element_type=jnp.float32)
    g = g / float(N)
    h1 = jnp.maximum(jnp.dot(g, rw1_ref[...]) + rb1_ref[...], 0.0)
    lg = jnp.dot(h1, rw2_ref[...]) + rb2_ref[...]
    lg = lg - jnp.max(lg, axis=-1, keepdims=True)
    ew = jnp.exp(lg)
    wts = ew / jnp.sum(ew, axis=-1, keepdims=True)
    sw = jnp.sum(wts, axis=-1, keepdims=True)
    wcolv = jnp.dot(mexp_ref[...], jnp.transpose(wts))
    S = (msel_ref[...] * wcolv).astype(jnp.bfloat16)

    ms = jnp.zeros((1, C), jnp.float32)
    mq = jnp.zeros((1, C), jnp.float32)
    for i in range(NSUB):
        yt = y_scr[i * MT:(i + 1) * MT, :].astype(jnp.float32)
        act = jnp.maximum(yt * A + Bc, 0.0).astype(jnp.bfloat16)
        acc = jnp.dot(act, S, preferred_element_type=jnp.float32)
        acc = acc + xb_ref[0, i * MT:(i + 1) * MT, :] * sw
        acc_scr[i * MT:(i + 1) * MT, :] = acc
        ms = ms + jnp.dot(ones_mt, acc, preferred_element_type=jnp.float32)
        mq = mq + jnp.dot(ones_mt, acc * acc,
                          preferred_element_type=jnp.float32)

    gs2 = jnp.dot(ms, m96_ref[...])
    gq2 = jnp.dot(mq, m96_ref[...])
    mu2 = gs2 / cnt
    var2 = gq2 / cnt - mu2 * mu2
    inv2 = jax.lax.rsqrt(var2 + EPS)
    mu2_c = jnp.dot(mu2, m8e_ref[...])
    inv2_c = jnp.dot(inv2, m8e_ref[...])
    A2 = inv2_c * mw_ref[...]
    B2 = mb_ref[...] - mu2_c * A2
    for i in range(NSUB):
        out_ref[0, i * MT:(i + 1) * MT, :] = (
            acc_scr[i * MT:(i + 1) * MT, :] * A2 + B2)
